# baseline (speedup 1.0000x reference)
import functools

import jax
import jax.numpy as jnp
from jax import lax
from jax.experimental import pallas as pl
from jax.experimental.pallas import tpu as pltpu
from jax.experimental.pallas import tpu_sc as plsc

_B, _N, _KA, _KB = 8, 1024, 16, 64
_BN = _B * _N
_EPS = 1e-5
_NW = 32
_RPW = _BN // _NW


def _bf(v):
    return v.astype(jnp.bfloat16).astype(jnp.float32)


def _k0_body(pos_ref, post_ref, w11_ref, b11_ref, w21_ref, b21_ref,
             w31_ref, b31_ref, idx_ref, h11_ref, h21_ref, h31_ref, d2_s):
    b = pl.program_id(0)
    px = pos_ref[:, 0:1]
    py = pos_ref[:, 1:2]
    pz = pos_ref[:, 2:3]
    qx = post_ref[0:1, :]
    qy = post_ref[1:2, :]
    qz = post_ref[2:3, :]
    pxb, pyb, pzb = _bf(px), _bf(py), _bf(pz)
    qxb, qyb, qzb = _bf(qx), _bf(qy), _bf(qz)
    sqc = px * px + py * py + pz * pz
    sqr = qx * qx + qy * qy + qz * qz
    d2 = sqc + sqr - 2.0 * (pxb * qxb + pyb * qyb + pzb * qzb)
    rows = lax.broadcasted_iota(jnp.int32, (_N, _N), 0)
    cols = lax.broadcasted_iota(jnp.int32, (_N, _N), 1)
    d2 = jnp.where(rows == cols, 1e10, d2)
    d2_s[...] = d2

    io64 = lax.broadcasted_iota(jnp.int32, (_N, _KB), 1)

    def extract(it, am_prev):
        d2c = jnp.where(cols == am_prev, 1e10, d2_s[...])
        d2_s[...] = d2c
        m = jnp.min(d2c, axis=1, keepdims=True)
        am = jnp.min(jnp.where(d2c <= m, cols, jnp.int32(2 ** 30)),
                     axis=1, keepdims=True)
        idx_ref[...] = jnp.where(io64 == it, am + b * _N, idx_ref[...])
        return am

    lax.fori_loop(0, _KB, extract,
                  jnp.full((_N, 1), -1, jnp.int32))

    for wr, br, hr in ((w11_ref, b11_ref, h11_ref),
                       (w21_ref, b21_ref, h21_ref),
                       (w31_ref, b31_ref, h31_ref)):
        hr[...] = (pxb * _bf(wr[0:1, :]) + pyb * _bf(wr[1:2, :])
                   + pzb * _bf(wr[2:3, :]) + br[...])


def _k0(pos, post, w11, b11, w21, b21, w31, b31):
    wspec = pl.BlockSpec((3, 32), lambda b: (0, 0))
    bspec = pl.BlockSpec((1, 32), lambda b: (0, 0))
    return pl.pallas_call(
        _k0_body,
        grid=(_B,),
        in_specs=[pl.BlockSpec((_N, 3), lambda b: (b, 0)),
                  pl.BlockSpec((3, _N), lambda b: (0, b)),
                  wspec, bspec, wspec, bspec, wspec, bspec],
        out_specs=[pl.BlockSpec((_N, _KB), lambda b: (b, 0)),
                   pl.BlockSpec((_N, 32), lambda b: (b, 0)),
                   pl.BlockSpec((_N, 32), lambda b: (b, 0)),
                   pl.BlockSpec((_N, 32), lambda b: (b, 0))],
        out_shape=[jax.ShapeDtypeStruct((_BN, _KB), jnp.int32),
                   jax.ShapeDtypeStruct((_BN, 32), jnp.float32),
                   jax.ShapeDtypeStruct((_BN, 32), jnp.float32),
                   jax.ShapeDtypeStruct((_BN, 32), jnp.float32)],
        scratch_shapes=[pltpu.VMEM((_N, _N), jnp.float32)],
    )(pos, post, w11, b11, w21, b21, w31, b31)


def _gather_max(x, idxf, k, f):
    rc = 128 // k
    nch = _RPW // rc
    chi = rc * k
    nf = f // 16
    mesh = plsc.VectorSubcoreMesh(core_axis_name="c", subcore_axis_name="s",
                                  num_cores=2)

    @functools.partial(
        pl.kernel,
        out_type=jax.ShapeDtypeStruct((_BN, f), jnp.float32),
        mesh=mesh,
        compiler_params=pltpu.CompilerParams(use_tc_tiling_on_sc=False),
        scratch_types=[pltpu.VMEM((_RPW * k,), jnp.int32),
                       pltpu.VMEM((chi, f), jnp.float32),
                       pltpu.VMEM((chi, f), jnp.float32),
                       pltpu.VMEM((_RPW, f), jnp.float32),
                       pltpu.SemaphoreType.DMA,
                       pltpu.SemaphoreType.DMA],
    )
    def body(x_hbm, idx_hbm, out_hbm, idx_v, g0, g1, out_v, s0, s1):
        cid = lax.axis_index("c")
        sid = lax.axis_index("s")
        wid = sid * 2 + cid
        base = wid * _RPW
        pltpu.sync_copy(idx_hbm.at[pl.ds(base * k, _RPW * k)], idx_v)

        def start(c, g, s):
            pltpu.async_copy(x_hbm.at[idx_v.at[pl.ds(c * chi, chi)]], g, s)

        def wait(g, s):
            pltpu.make_async_copy(
                x_hbm.at[idx_v.at[pl.ds(0, chi)]], g, s).wait()

        def compute(c, g):
            def row(r, _):
                rb = r * k

                def red(j, acc):
                    return tuple(
                        jnp.maximum(acc[q], g[rb + j, pl.ds(q * 16, 16)])
                        for q in range(nf))

                acc = lax.fori_loop(
                    1, k, red,
                    tuple(g[rb, pl.ds(q * 16, 16)] for q in range(nf)),
                    unroll=4)
                for q in range(nf):
                    out_v[c * rc + r, pl.ds(q * 16, 16)] = acc[q]
                return 0

            lax.fori_loop(0, rc, row, 0)

        start(0, g0, s0)
        start(1, g1, s1)

        def step(i, _):
            c2 = i * 2
            wait(g0, s0)
            compute(c2, g0)

            @pl.when(c2 + 2 < nch)
            def _():
                start(c2 + 2, g0, s0)

            wait(g1, s1)
            compute(c2 + 1, g1)

            @pl.when(c2 + 3 < nch)
            def _():
                start(c2 + 3, g1, s1)

            return 0

        lax.fori_loop(0, nch // 2, step, 0)
        pltpu.sync_copy(out_v, out_hbm.at[pl.ds(base, _RPW)])

    return body(x, idxf)


def _bn_relu(x, g, be):
    m = jnp.mean(x, axis=0, keepdims=True)
    xc = x - m
    v = jnp.mean(xc * xc, axis=0, keepdims=True)
    return jnp.maximum(xc * lax.rsqrt(v + _EPS) * g + be, 0.0)


def _dot(a, b):
    return jnp.dot(a.astype(jnp.bfloat16), b.astype(jnp.bfloat16),
                   preferred_element_type=jnp.float32)


def _k1_body(pos, a11, a21, h31, w12p, w12b, b12, w22p, w22b, b22, w32, b32,
             g11, be11, g21, be21, g31, be31, g32, be32,
             x1bn_o, x2bn_o, h12_o, h22_o, x3_o):
    pxb = _bf(pos[:, 0:1])
    pyb = _bf(pos[:, 1:2])
    pzb = _bf(pos[:, 2:3])
    x1bn = _bn_relu(a11[...], g11[...], be11[...])
    x1bn_o[...] = x1bn
    x2bn = _bn_relu(a21[...], g21[...], be21[...])
    x2bn_o[...] = x2bn
    h12_o[...] = (pxb * _bf(w12p[0:1, :]) + pyb * _bf(w12p[1:2, :])
                  + pzb * _bf(w12p[2:3, :])
                  + _dot(x1bn, w12b[...]) + b12[...])
    h22_o[...] = (pxb * _bf(w22p[0:1, :]) + pyb * _bf(w22p[1:2, :])
                  + pzb * _bf(w22p[2:3, :])
                  + _dot(x2bn, w22b[...]) + b22[...])
    x31 = _bn_relu(h31[...], g31[...], be31[...])
    h32 = _dot(x31, w32[...]) + b32[...]
    x3_o[...] = _bn_relu(h32, g32[...], be32[...])


def _k1(pos, a11, a21, h31, args):
    return pl.pallas_call(
        _k1_body,
        out_shape=[jax.ShapeDtypeStruct((_BN, 32), jnp.float32),
                   jax.ShapeDtypeStruct((_BN, 32), jnp.float32),
                   jax.ShapeDtypeStruct((_BN, 128), jnp.float32),
                   jax.ShapeDtypeStruct((_BN, 128), jnp.float32),
                   jax.ShapeDtypeStruct((_BN, 128), jnp.float32)],
    )(pos, a11, a21, h31, *args)


def _k2a_body(a12, a22, g12, be12, g22, be22, x1bn2_o, x2bn2_o):
    x1bn2_o[...] = _bn_relu(a12[...], g12[...], be12[...])
    x2bn2_o[...] = _bn_relu(a22[...], g22[...], be22[...])


def _k2b_body(pos, x1bn, x1bn2, x2bn, x2bn2, x3, wp, w1, w2, w3, w4, w5,
              bfa, t_o):
    pxb = _bf(pos[:, 0:1])
    pyb = _bf(pos[:, 1:2])
    pzb = _bf(pos[:, 2:3])
    t_o[...] = (
        pxb * _bf(wp[0:1, :]) + pyb * _bf(wp[1:2, :]) + pzb * _bf(wp[2:3, :])
        + pxb * _bf(wp[3:4, :]) + pyb * _bf(wp[4:5, :]) + pzb * _bf(wp[5:6, :])
        + _dot(x1bn[...], w1[...]) + _dot(x1bn2[...], w2[...])
        + _dot(x2bn[...], w3[...]) + _dot(x2bn2[...], w4[...])
        + _dot(x3[...], w5[...]) + bfa[...])


def _k2c_body(t, gfa, befa, wout, bout, out_o):
    y = _bn_relu(t[...], gfa[...], befa[...])
    pool = jnp.max(y.reshape(_B, _N, 256), axis=1)
    out_o[...] = _dot(pool, wout[...]) + bout[...]


def _k2(pos, x1bn, a12, x2bn, a22, x3, args):
    (wp, w1, w2, w3, w4, w5, bfa, g12, be12, g22, be22, gfa, befa,
     wout, bout) = args
    x1bn2, x2bn2 = pl.pallas_call(
        _k2a_body,
        out_shape=[jax.ShapeDtypeStruct((_BN, 128), jnp.float32)] * 2,
    )(a12, a22, g12, be12, g22, be22)
    full = lambda s: pl.BlockSpec(s, lambda b: (0, 0))
    t = pl.pallas_call(
        _k2b_body,
        grid=(_B,),
        in_specs=[pl.BlockSpec((_N, 3), lambda b: (b, 0)),
                  pl.BlockSpec((_N, 32), lambda b: (b, 0)),
                  pl.BlockSpec((_N, 128), lambda b: (b, 0)),
                  pl.BlockSpec((_N, 32), lambda b: (b, 0)),
                  pl.BlockSpec((_N, 128), lambda b: (b, 0)),
                  pl.BlockSpec((_N, 128), lambda b: (b, 0)),
                  full((6, 256)), full((32, 256)), full((128, 256)),
                  full((32, 256)), full((128, 256)), full((128, 256)),
                  full((1, 256))],
        out_specs=pl.BlockSpec((_N, 256), lambda b: (b, 0)),
        out_shape=jax.ShapeDtypeStruct((_BN, 256), jnp.float32),
    )(pos, x1bn, x1bn2, x2bn, x2bn2, x3, wp, w1, w2, w3, w4, w5, bfa)
    return pl.pallas_call(
        _k2c_body,
        out_shape=jax.ShapeDtypeStruct((_B, 40), jnp.float32),
    )(t, gfa, befa, wout, bout)


def kernel(pos, batch, params):
    p = params
    r2 = lambda a: a.reshape(1, -1)

    idx64, h11, h21, h31 = _k0(
        pos, pos.T,
        p['bn1_1']['W'], r2(p['bn1_1']['b']),
        p['bn2_1']['W'], r2(p['bn2_1']['b']),
        p['bn3_1']['W'], r2(p['bn3_1']['b']))
    idx16f = idx64[:, :_KA].reshape(-1)
    idx64f = idx64.reshape(-1)

    a11 = _gather_max(h11, idx16f, _KA, 32)
    a21 = _gather_max(h21, idx64f, _KB, 32)

    w12 = p['bn1_2']['W']
    w22 = p['bn2_2']['W']
    k1_args = (w12[:3], w12[3:], r2(p['bn1_2']['b']),
               w22[:3], w22[3:], r2(p['bn2_2']['b']),
               p['bn3_2']['W'], r2(p['bn3_2']['b']),
               r2(p['bn1_1']['gamma']), r2(p['bn1_1']['beta']),
               r2(p['bn2_1']['gamma']), r2(p['bn2_1']['beta']),
               r2(p['bn3_1']['gamma']), r2(p['bn3_1']['beta']),
               r2(p['bn3_2']['gamma']), r2(p['bn3_2']['beta']))
    x1bn, x2bn, h12, h22, x3 = _k1(pos, a11, a21, h31, k1_args)

    a12 = _gather_max(h12, idx16f, _KA, 128)
    a22 = _gather_max(h22, idx64f, _KB, 128)

    wfa = p['fa']['W']
    k2_args = (jnp.concatenate([wfa[0:3], wfa[163:166]]), wfa[3:35], wfa[35:163],
               wfa[166:198], wfa[198:326], wfa[326:454], r2(p['fa']['b']),
               r2(p['bn1_2']['gamma']), r2(p['bn1_2']['beta']),
               r2(p['bn2_2']['gamma']), r2(p['bn2_2']['beta']),
               r2(p['fa']['gamma']), r2(p['fa']['beta']),
               p['out']['W'], r2(p['out']['b']))
    return _k2(pos, x1bn, a12, x2bn, a22, x3, k2_args)


# speedup vs baseline: 1.0326x; 1.0326x over previous
import functools

import jax
import jax.numpy as jnp
from jax import lax
from jax.experimental import pallas as pl
from jax.experimental.pallas import tpu as pltpu
from jax.experimental.pallas import tpu_sc as plsc

_B, _N, _KA, _KB = 8, 1024, 16, 64
_BN = _B * _N
_EPS = 1e-5
_NW = 32
_RPW = _BN // _NW


def _bf(v):
    return v.astype(jnp.bfloat16).astype(jnp.float32)


def _k0_body(pos_ref, post_ref, w11_ref, b11_ref, w21_ref, b21_ref,
             w31_ref, b31_ref, idx_ref, h11_ref, h21_ref, h31_ref, d2_s):
    b = pl.program_id(0)
    px = pos_ref[:, 0:1]
    py = pos_ref[:, 1:2]
    pz = pos_ref[:, 2:3]
    qx = post_ref[0:1, :]
    qy = post_ref[1:2, :]
    qz = post_ref[2:3, :]
    pxb, pyb, pzb = _bf(px), _bf(py), _bf(pz)
    qxb, qyb, qzb = _bf(qx), _bf(qy), _bf(qz)
    sqc = px * px + py * py + pz * pz
    sqr = qx * qx + qy * qy + qz * qz
    d2 = sqc + sqr - 2.0 * (pxb * qxb + pyb * qyb + pzb * qzb)
    rows = lax.broadcasted_iota(jnp.int32, (_N, _N), 0)
    cols = lax.broadcasted_iota(jnp.int32, (_N, _N), 1)
    d2 = jnp.where(rows == cols, 1e10, d2)
    d2_s[...] = d2

    io64 = lax.broadcasted_iota(jnp.int32, (_N, _KB), 1)

    def extract(it, _):
        d2c = d2_s[...]
        m = jnp.min(d2c, axis=1, keepdims=True)
        am = jnp.min(jnp.where(d2c <= m, cols, jnp.int32(2 ** 30)),
                     axis=1, keepdims=True)
        idx_ref[...] = jnp.where(io64 == it, am + b * _N, idx_ref[...])
        d2_s[...] = jnp.where(cols == am, 1e10, d2c)
        return 0

    lax.fori_loop(0, _KB, extract, 0)

    for wr, br, hr in ((w11_ref, b11_ref, h11_ref),
                       (w21_ref, b21_ref, h21_ref),
                       (w31_ref, b31_ref, h31_ref)):
        hr[...] = (pxb * _bf(wr[0:1, :]) + pyb * _bf(wr[1:2, :])
                   + pzb * _bf(wr[2:3, :]) + br[...])


def _k0(pos, post, w11, b11, w21, b21, w31, b31):
    wspec = pl.BlockSpec((3, 32), lambda b: (0, 0))
    bspec = pl.BlockSpec((1, 32), lambda b: (0, 0))
    return pl.pallas_call(
        _k0_body,
        grid=(_B,),
        in_specs=[pl.BlockSpec((_N, 3), lambda b: (b, 0)),
                  pl.BlockSpec((3, _N), lambda b: (0, b)),
                  wspec, bspec, wspec, bspec, wspec, bspec],
        out_specs=[pl.BlockSpec((_N, _KB), lambda b: (b, 0)),
                   pl.BlockSpec((_N, 32), lambda b: (b, 0)),
                   pl.BlockSpec((_N, 32), lambda b: (b, 0)),
                   pl.BlockSpec((_N, 32), lambda b: (b, 0))],
        out_shape=[jax.ShapeDtypeStruct((_BN, _KB), jnp.int32),
                   jax.ShapeDtypeStruct((_BN, 32), jnp.float32),
                   jax.ShapeDtypeStruct((_BN, 32), jnp.float32),
                   jax.ShapeDtypeStruct((_BN, 32), jnp.float32)],
        scratch_shapes=[pltpu.VMEM((_N, _N), jnp.float32)],
    )(pos, post, w11, b11, w21, b21, w31, b31)


def _gather_max(x, idxf, k, f):
    rc = 128 // k
    nch = _RPW // rc
    chi = rc * k
    nf = f // 16
    mesh = plsc.VectorSubcoreMesh(core_axis_name="c", subcore_axis_name="s",
                                  num_cores=2)

    @functools.partial(
        pl.kernel,
        out_type=jax.ShapeDtypeStruct((_BN, f), jnp.float32),
        mesh=mesh,
        compiler_params=pltpu.CompilerParams(use_tc_tiling_on_sc=False),
        scratch_types=[pltpu.VMEM((_RPW * k,), jnp.int32),
                       pltpu.VMEM((chi, f), jnp.float32),
                       pltpu.VMEM((chi, f), jnp.float32),
                       pltpu.VMEM((_RPW, f), jnp.float32),
                       pltpu.SemaphoreType.DMA,
                       pltpu.SemaphoreType.DMA],
    )
    def body(x_hbm, idx_hbm, out_hbm, idx_v, g0, g1, out_v, s0, s1):
        cid = lax.axis_index("c")
        sid = lax.axis_index("s")
        wid = sid * 2 + cid
        base = wid * _RPW
        pltpu.sync_copy(idx_hbm.at[pl.ds(base * k, _RPW * k)], idx_v)

        def start(c, g, s):
            pltpu.async_copy(x_hbm.at[idx_v.at[pl.ds(c * chi, chi)]], g, s)

        def wait(g, s):
            pltpu.make_async_copy(
                x_hbm.at[idx_v.at[pl.ds(0, chi)]], g, s).wait()

        def compute(c, g):
            def row(r, _):
                rb = r * k

                def red(j, acc):
                    return tuple(
                        jnp.maximum(acc[q], g[rb + j, pl.ds(q * 16, 16)])
                        for q in range(nf))

                acc = lax.fori_loop(
                    1, k, red,
                    tuple(g[rb, pl.ds(q * 16, 16)] for q in range(nf)),
                    unroll=8 if k == _KB else 5)
                for q in range(nf):
                    out_v[c * rc + r, pl.ds(q * 16, 16)] = acc[q]
                return 0

            lax.fori_loop(0, rc, row, 0)

        start(0, g0, s0)
        start(1, g1, s1)

        def step(i, _):
            c2 = i * 2
            wait(g0, s0)
            compute(c2, g0)

            @pl.when(c2 + 2 < nch)
            def _():
                start(c2 + 2, g0, s0)

            wait(g1, s1)
            compute(c2 + 1, g1)

            @pl.when(c2 + 3 < nch)
            def _():
                start(c2 + 3, g1, s1)

            return 0

        lax.fori_loop(0, nch // 2, step, 0)
        pltpu.sync_copy(out_v, out_hbm.at[pl.ds(base, _RPW)])

    return body(x, idxf)


def _bn_relu(x, g, be):
    m = jnp.mean(x, axis=0, keepdims=True)
    xc = x - m
    v = jnp.mean(xc * xc, axis=0, keepdims=True)
    return jnp.maximum(xc * lax.rsqrt(v + _EPS) * g + be, 0.0)


def _dot(a, b):
    return jnp.dot(a.astype(jnp.bfloat16), b.astype(jnp.bfloat16),
                   preferred_element_type=jnp.float32)


def _k1_body(pos, a11, a21, h31, w12p, w12b, b12, w22p, w22b, b22, w32, b32,
             g11, be11, g21, be21, g31, be31, g32, be32,
             x1bn_o, x2bn_o, h12_o, h22_o, x3_o):
    pxb = _bf(pos[:, 0:1])
    pyb = _bf(pos[:, 1:2])
    pzb = _bf(pos[:, 2:3])
    x1bn = _bn_relu(a11[...], g11[...], be11[...])
    x1bn_o[...] = x1bn
    x2bn = _bn_relu(a21[...], g21[...], be21[...])
    x2bn_o[...] = x2bn
    h12_o[...] = (pxb * _bf(w12p[0:1, :]) + pyb * _bf(w12p[1:2, :])
                  + pzb * _bf(w12p[2:3, :])
                  + _dot(x1bn, w12b[...]) + b12[...])
    h22_o[...] = (pxb * _bf(w22p[0:1, :]) + pyb * _bf(w22p[1:2, :])
                  + pzb * _bf(w22p[2:3, :])
                  + _dot(x2bn, w22b[...]) + b22[...])
    x31 = _bn_relu(h31[...], g31[...], be31[...])
    h32 = _dot(x31, w32[...]) + b32[...]
    x3_o[...] = _bn_relu(h32, g32[...], be32[...])


def _k1(pos, a11, a21, h31, args):
    return pl.pallas_call(
        _k1_body,
        out_shape=[jax.ShapeDtypeStruct((_BN, 32), jnp.float32),
                   jax.ShapeDtypeStruct((_BN, 32), jnp.float32),
                   jax.ShapeDtypeStruct((_BN, 128), jnp.float32),
                   jax.ShapeDtypeStruct((_BN, 128), jnp.float32),
                   jax.ShapeDtypeStruct((_BN, 128), jnp.float32)],
    )(pos, a11, a21, h31, *args)


def _k2a_body(a12, a22, g12, be12, g22, be22, x1bn2_o, x2bn2_o):
    x1bn2_o[...] = _bn_relu(a12[...], g12[...], be12[...])
    x2bn2_o[...] = _bn_relu(a22[...], g22[...], be22[...])


def _k2b_body(pos, x1bn, x1bn2, x2bn, x2bn2, x3, wp, w1, w2, w3, w4, w5,
              bfa, t_o):
    pxb = _bf(pos[:, 0:1])
    pyb = _bf(pos[:, 1:2])
    pzb = _bf(pos[:, 2:3])
    t_o[...] = (
        pxb * _bf(wp[0:1, :]) + pyb * _bf(wp[1:2, :]) + pzb * _bf(wp[2:3, :])
        + pxb * _bf(wp[3:4, :]) + pyb * _bf(wp[4:5, :]) + pzb * _bf(wp[5:6, :])
        + _dot(x1bn[...], w1[...]) + _dot(x1bn2[...], w2[...])
        + _dot(x2bn[...], w3[...]) + _dot(x2bn2[...], w4[...])
        + _dot(x3[...], w5[...]) + bfa[...])


def _k2c_body(t, gfa, befa, wout, bout, out_o):
    y = _bn_relu(t[...], gfa[...], befa[...])
    pool = jnp.max(y.reshape(_B, _N, 256), axis=1)
    out_o[...] = _dot(pool, wout[...]) + bout[...]


def _k2(pos, x1bn, a12, x2bn, a22, x3, args):
    (wp, w1, w2, w3, w4, w5, bfa, g12, be12, g22, be22, gfa, befa,
     wout, bout) = args
    x1bn2, x2bn2 = pl.pallas_call(
        _k2a_body,
        out_shape=[jax.ShapeDtypeStruct((_BN, 128), jnp.float32)] * 2,
    )(a12, a22, g12, be12, g22, be22)
    full = lambda s: pl.BlockSpec(s, lambda b: (0, 0))
    t = pl.pallas_call(
        _k2b_body,
        grid=(_B,),
        in_specs=[pl.BlockSpec((_N, 3), lambda b: (b, 0)),
                  pl.BlockSpec((_N, 32), lambda b: (b, 0)),
                  pl.BlockSpec((_N, 128), lambda b: (b, 0)),
                  pl.BlockSpec((_N, 32), lambda b: (b, 0)),
                  pl.BlockSpec((_N, 128), lambda b: (b, 0)),
                  pl.BlockSpec((_N, 128), lambda b: (b, 0)),
                  full((6, 256)), full((32, 256)), full((128, 256)),
                  full((32, 256)), full((128, 256)), full((128, 256)),
                  full((1, 256))],
        out_specs=pl.BlockSpec((_N, 256), lambda b: (b, 0)),
        out_shape=jax.ShapeDtypeStruct((_BN, 256), jnp.float32),
    )(pos, x1bn, x1bn2, x2bn, x2bn2, x3, wp, w1, w2, w3, w4, w5, bfa)
    return pl.pallas_call(
        _k2c_body,
        out_shape=jax.ShapeDtypeStruct((_B, 40), jnp.float32),
    )(t, gfa, befa, wout, bout)


def kernel(pos, batch, params):
    p = params
    r2 = lambda a: a.reshape(1, -1)

    idx64, h11, h21, h31 = _k0(
        pos, pos.T,
        p['bn1_1']['W'], r2(p['bn1_1']['b']),
        p['bn2_1']['W'], r2(p['bn2_1']['b']),
        p['bn3_1']['W'], r2(p['bn3_1']['b']))
    idx16f = idx64[:, :_KA].reshape(-1)
    idx64f = idx64.reshape(-1)

    a11 = _gather_max(h11, idx16f, _KA, 32)
    a21 = _gather_max(h21, idx64f, _KB, 32)

    w12 = p['bn1_2']['W']
    w22 = p['bn2_2']['W']
    k1_args = (w12[:3], w12[3:], r2(p['bn1_2']['b']),
               w22[:3], w22[3:], r2(p['bn2_2']['b']),
               p['bn3_2']['W'], r2(p['bn3_2']['b']),
               r2(p['bn1_1']['gamma']), r2(p['bn1_1']['beta']),
               r2(p['bn2_1']['gamma']), r2(p['bn2_1']['beta']),
               r2(p['bn3_1']['gamma']), r2(p['bn3_1']['beta']),
               r2(p['bn3_2']['gamma']), r2(p['bn3_2']['beta']))
    x1bn, x2bn, h12, h22, x3 = _k1(pos, a11, a21, h31, k1_args)

    a12 = _gather_max(h12, idx16f, _KA, 128)
    a22 = _gather_max(h22, idx64f, _KB, 128)

    wfa = p['fa']['W']
    k2_args = (jnp.concatenate([wfa[0:3], wfa[163:166]]), wfa[3:35], wfa[35:163],
               wfa[166:198], wfa[198:326], wfa[326:454], r2(p['fa']['b']),
               r2(p['bn1_2']['gamma']), r2(p['bn1_2']['beta']),
               r2(p['bn2_2']['gamma']), r2(p['bn2_2']['beta']),
               r2(p['fa']['gamma']), r2(p['fa']['beta']),
               p['out']['W'], r2(p['out']['b']))
    return _k2(pos, x1bn, a12, x2bn, a22, x3, k2_args)
